# Initial kernel scaffold; baseline (speedup 1.0000x reference)
import jax
import jax.numpy as jnp
from jax.experimental import pallas as pl


def kernel(pos_target, pos_contexts, pos_negatives, context_table, output_table):
    raise NotImplementedError("write your pallas kernel here")


# speedup vs baseline: 7.0142x; 7.0142x over previous
import functools

import jax
import jax.numpy as jnp
from jax import lax
from jax.experimental import pallas as pl
from jax.experimental.pallas import tpu as pltpu
from jax.experimental.pallas import tpu_sc as plsc

VOCAB = 100000
DIM = 64
B = 16384
N_CTX = 50
N_NEG = 20
N_TOT = N_NEG + 1

L = 16
NV = DIM // L
NC, NS = 2, 16
NW = NC * NS
B_PER_W = B // NW
C = 8
N_CHUNK = B_PER_W // C


def _sc_body(ctx_idx_hbm, out_idx_hbm, ctx_tab_hbm, out_tab_hbm, part_hbm,
             ctx_idx_v, out_idx_v, ctx_rows_v, out_rows_v, part_v, sem):
    c = lax.axis_index("c")
    s = lax.axis_index("s")
    wid = s * NC + c
    base = wid * B_PER_W

    def chunk_body(g, carry):
        b0 = base + g * C
        pltpu.sync_copy(ctx_idx_hbm.at[pl.ds(b0, C)], ctx_idx_v)
        pltpu.sync_copy(out_idx_hbm.at[pl.ds(b0, C)], out_idx_v)
        for i in range(C):
            pltpu.async_copy(ctx_tab_hbm.at[ctx_idx_v.at[i]], ctx_rows_v.at[i], sem)
            pltpu.async_copy(out_tab_hbm.at[out_idx_v.at[i]], out_rows_v.at[i], sem)
        for i in range(C):
            pltpu.make_async_copy(ctx_tab_hbm.at[ctx_idx_v.at[i]], ctx_rows_v.at[i], sem).wait()
            pltpu.make_async_copy(out_tab_hbm.at[out_idx_v.at[i]], out_rows_v.at[i], sem).wait()

        for i in range(C):
            def sum_body(r, accs):
                return tuple(accs[k] + ctx_rows_v[i, r, pl.ds(k * L, L)]
                             for k in range(NV))
            accs = lax.fori_loop(
                0, N_CTX, sum_body,
                tuple(jnp.zeros((L,), jnp.float32) for _ in range(NV)))

            def dot_body(j, carry2):
                p = accs[0] * out_rows_v[i, j, pl.ds(0, L)]
                for k in range(1, NV):
                    p = p + accs[k] * out_rows_v[i, j, pl.ds(k * L, L)]
                part_v[i, j] = p
                return carry2
            lax.fori_loop(0, N_TOT, dot_body, 0)

        pltpu.sync_copy(part_v, part_hbm.at[pl.ds(b0, C)])
        return carry

    lax.fori_loop(0, N_CHUNK, chunk_body, 0)


@jax.jit
def _sc_gather_dot(ctx_idx, out_idx, ctx_tab, out_tab):
    mesh = plsc.VectorSubcoreMesh(core_axis_name="c", subcore_axis_name="s")
    return pl.kernel(
        _sc_body,
        out_type=jax.ShapeDtypeStruct((B, N_TOT, L), jnp.float32),
        mesh=mesh,
        compiler_params=pltpu.CompilerParams(use_tc_tiling_on_sc=False),
        scratch_types=[
            pltpu.VMEM((C, N_CTX), jnp.int32),
            pltpu.VMEM((C, N_TOT), jnp.int32),
            pltpu.VMEM((C, N_CTX, DIM), jnp.float32),
            pltpu.VMEM((C, N_TOT, DIM), jnp.float32),
            pltpu.VMEM((C, N_TOT, L), jnp.float32),
            pltpu.SemaphoreType.DMA,
        ],
    )(ctx_idx, out_idx, ctx_tab, out_tab)


_TC_BLK = 2048


def _tc_loss_body(p_ref, o_ref):
    pid = pl.program_id(0)
    p = p_ref[...]
    rows = lax.broadcasted_iota(jnp.int32, (N_TOT * L, N_TOT), 0) // L
    cols = lax.broadcasted_iota(jnp.int32, (N_TOT * L, N_TOT), 1)
    sel = (rows == cols).astype(jnp.float32)
    scores = jnp.dot(p, sel, preferred_element_type=jnp.float32)
    scores = jnp.clip(scores, -10.0, 10.0)
    sgn = jnp.where(
        lax.broadcasted_iota(jnp.int32, scores.shape, 1) == 0, -1.0, 1.0)
    terms = jax.nn.softplus(sgn * scores)
    blk_sum = jnp.sum(terms)

    @pl.when(pid == 0)
    def _():
        o_ref[0, 0] = 0.0

    o_ref[0, 0] += blk_sum


@jax.jit
def _tc_loss(part2d):
    out = pl.pallas_call(
        _tc_loss_body,
        grid=(B // _TC_BLK,),
        in_specs=[pl.BlockSpec((_TC_BLK, N_TOT * L), lambda i: (i, 0))],
        out_specs=pl.BlockSpec(memory_space=pltpu.SMEM),
        out_shape=jax.ShapeDtypeStruct((1, 1), jnp.float32),
    )(part2d)
    return out[0, 0] / B


def kernel(pos_target, pos_contexts, pos_negatives, context_table, output_table):
    ctx_idx = pos_contexts.astype(jnp.int32)
    out_idx = jnp.concatenate(
        [pos_target[:, None], pos_negatives], axis=1).astype(jnp.int32)
    part = _sc_gather_dot(ctx_idx, out_idx, context_table, output_table)
    return _tc_loss(part.reshape(B, N_TOT * L))


# speedup vs baseline: 9.7473x; 1.3896x over previous
import functools

import jax
import jax.numpy as jnp
from jax import lax
from jax.experimental import pallas as pl
from jax.experimental.pallas import tpu as pltpu
from jax.experimental.pallas import tpu_sc as plsc

VOCAB = 100000
DIM = 64
B = 16384
N_CTX = 50
N_NEG = 20
N_TOT = N_NEG + 1

L = 16
NV = DIM // L
NC, NS = 2, 16
NW = NC * NS
B_PER_W = B // NW
C = 8
N_CHUNK = B_PER_W // C


def _sc_body(ctx_idx_hbm, out_idx_hbm, ctx_tab_hbm, out_tab_hbm, part_hbm,
             ctx_idx_v, out_idx_v, ctx_rows_v, out_rows_v, part_v,
             sem_r0, sem_r1, sem_p0, sem_p1):
    c = lax.axis_index("c")
    s = lax.axis_index("s")
    wid = s * NC + c
    base = wid * B_PER_W
    sem_r = (sem_r0, sem_r1)
    sem_p = (sem_p0, sem_p1)

    pltpu.sync_copy(ctx_idx_hbm.at[pl.ds(base * N_CTX, B_PER_W * N_CTX)],
                    ctx_idx_v)
    pltpu.sync_copy(out_idx_hbm.at[pl.ds(base * N_TOT, B_PER_W * N_TOT)],
                    out_idx_v)

    def row_copies(slot, g):
        return (
            pltpu.make_async_copy(
                ctx_tab_hbm.at[ctx_idx_v.at[pl.ds(g * (C * N_CTX), C * N_CTX)]],
                ctx_rows_v.at[slot], sem_r[slot]),
            pltpu.make_async_copy(
                out_tab_hbm.at[out_idx_v.at[pl.ds(g * (C * N_TOT), C * N_TOT)]],
                out_rows_v.at[slot], sem_r[slot]),
        )

    def part_copy(slot, g):
        return pltpu.make_async_copy(
            part_v.at[slot], part_hbm.at[pl.ds(base + g * C, C)], sem_p[slot])

    for cp in row_copies(0, 0):
        cp.start()

    def loop_body(g2, carry):
        for p in (0, 1):
            g = g2 * 2 + p
            q = 1 - p

            @pl.when(g + 1 < N_CHUNK)
            def _():
                for cp in row_copies(q, g + 1):
                    cp.start()

            for cp in row_copies(p, g):
                cp.wait()

            @pl.when(g >= 2)
            def _():
                part_copy(p, g).wait()

            for i in range(C):
                def sum_body(r, accs):
                    return tuple(
                        accs[k] + ctx_rows_v[p, i * N_CTX + r, pl.ds(k * L, L)]
                        for k in range(NV))
                accs = lax.fori_loop(
                    0, N_CTX, sum_body,
                    tuple(jnp.zeros((L,), jnp.float32) for _ in range(NV)))

                def dot_body(j, carry2):
                    pr = accs[0] * out_rows_v[p, i * N_TOT + j, pl.ds(0, L)]
                    for k in range(1, NV):
                        pr = pr + accs[k] * out_rows_v[p, i * N_TOT + j,
                                                       pl.ds(k * L, L)]
                    part_v[p, i, j] = pr
                    return carry2
                lax.fori_loop(0, N_TOT, dot_body, 0)

            part_copy(p, g).start()
        return carry

    lax.fori_loop(0, N_CHUNK // 2, loop_body, 0)
    part_copy(0, N_CHUNK - 2).wait()
    part_copy(1, N_CHUNK - 1).wait()


@jax.jit
def _sc_gather_dot(ctx_idx, out_idx, ctx_tab, out_tab):
    mesh = plsc.VectorSubcoreMesh(core_axis_name="c", subcore_axis_name="s")
    return pl.kernel(
        _sc_body,
        out_type=jax.ShapeDtypeStruct((B, N_TOT, L), jnp.float32),
        mesh=mesh,
        compiler_params=pltpu.CompilerParams(use_tc_tiling_on_sc=False),
        scratch_types=[
            pltpu.VMEM((B_PER_W * N_CTX,), jnp.int32),
            pltpu.VMEM((B_PER_W * N_TOT,), jnp.int32),
            pltpu.VMEM((2, C * N_CTX, DIM), jnp.float32),
            pltpu.VMEM((2, C * N_TOT, DIM), jnp.float32),
            pltpu.VMEM((2, C, N_TOT, L), jnp.float32),
            pltpu.SemaphoreType.DMA,
            pltpu.SemaphoreType.DMA,
            pltpu.SemaphoreType.DMA,
            pltpu.SemaphoreType.DMA,
        ],
    )(ctx_idx, out_idx, ctx_tab, out_tab)


_TC_BLK = 2048


def _tc_loss_body(p_ref, o_ref):
    pid = pl.program_id(0)
    p = p_ref[...]
    rows = lax.broadcasted_iota(jnp.int32, (N_TOT * L, N_TOT), 0) // L
    cols = lax.broadcasted_iota(jnp.int32, (N_TOT * L, N_TOT), 1)
    sel = (rows == cols).astype(jnp.float32)
    scores = jnp.dot(p, sel, preferred_element_type=jnp.float32)
    scores = jnp.clip(scores, -10.0, 10.0)
    sgn = jnp.where(
        lax.broadcasted_iota(jnp.int32, scores.shape, 1) == 0, -1.0, 1.0)
    terms = jax.nn.softplus(sgn * scores)
    blk_sum = jnp.sum(terms)

    @pl.when(pid == 0)
    def _():
        o_ref[0, 0] = 0.0

    o_ref[0, 0] += blk_sum


@jax.jit
def _tc_loss(part2d):
    out = pl.pallas_call(
        _tc_loss_body,
        grid=(B // _TC_BLK,),
        in_specs=[pl.BlockSpec((_TC_BLK, N_TOT * L), lambda i: (i, 0))],
        out_specs=pl.BlockSpec(memory_space=pltpu.SMEM),
        out_shape=jax.ShapeDtypeStruct((1, 1), jnp.float32),
    )(part2d)
    return out[0, 0] / B


def kernel(pos_target, pos_contexts, pos_negatives, context_table, output_table):
    ctx_idx = pos_contexts.astype(jnp.int32).reshape(B * N_CTX)
    out_idx = jnp.concatenate(
        [pos_target[:, None], pos_negatives],
        axis=1).astype(jnp.int32).reshape(B * N_TOT)
    part = _sc_gather_dot(ctx_idx, out_idx, context_table, output_table)
    return _tc_loss(part.reshape(B, N_TOT * L))
